# baseline (speedup 1.0000x reference)
import functools

import jax
import jax.numpy as jnp
from jax import lax
from jax.experimental import pallas as pl
from jax.experimental.pallas import tpu as pltpu
from jax.experimental.pallas import tpu_sc as plsc

_EPS = 1e-5
_N, _V, _D = 16384, 100000, 128
_NC, _NS, _L = 2, 16, 16
_NW = _NC * _NS
_CHUNK = _N // _NW
_SLICE = _N // _NS
_CB = _CHUNK + _L
_NB = _CB // _L
_TOT = _CHUNK + 2 * _L


def _sc_body(values_hbm, numbers_hbm, table_hbm, gamma_hbm, beta_hbm, out_hbm,
             vals_v, nums_v, rows_v, cidx, cposa, cposp, cx,
             aidx2, aposa2, aposp2, gb_v, pack_v, all_v, shared,
             gsem, osem, ssem):
    cid = lax.axis_index("c")
    sid = lax.axis_index("s")
    wid = sid * _NC + cid
    base = wid * _CHUNK

    with jax.named_scope("stage"):
        vals_cp = pltpu.async_copy(values_hbm.at[pl.ds(base, _CHUNK)],
                                   vals_v, ssem.at[0])
        num_cp = pltpu.async_copy(numbers_hbm.at[pl.ds(sid * _SLICE, _SLICE)],
                                  nums_v, ssem.at[1])
        g_cp = pltpu.async_copy(gamma_hbm, gb_v.at[pl.ds(0, 1)], ssem.at[1])
        b_cp = pltpu.async_copy(beta_hbm, gb_v.at[pl.ds(8, 1)], ssem.at[1])
        num_cp.wait()
        g_cp.wait()
        b_cp.wait()

    lane = lax.iota(jnp.int32, _L)

    def allsum(x):
        for k in (1, 2, 4, 8):
            x = x + x.at[lane ^ k].get(mode="promise_in_bounds")
        return x

    def stats_step(i, carry):
        s, ss, cnt = carry
        for u in range(8):
            x = nums_v[pl.ds(i * 8 * _L + u * _L, _L)]
            pres = x == x
            xs = jnp.where(pres, x, 0.0)
            s = s + xs
            ss = ss + xs * xs
            cnt = cnt + jnp.where(pres, 1.0, 0.0)
        return s, ss, cnt

    with jax.named_scope("stats"):
        zero = jnp.zeros((_L,), jnp.float32)
        s, ss, cnt = lax.fori_loop(0, _SLICE // (8 * _L), stats_step,
                                   (zero, zero, zero))
        pack = jnp.where(lane == 0, allsum(s),
                         jnp.where(lane == 1, allsum(ss), allsum(cnt)))
        pack_v[pl.ds(0, _L)] = pack
        pltpu.sync_copy(pack_v, shared.at[pl.ds(sid * _L, _L)])

    coff = cid * _CHUNK
    with jax.named_scope("compact"):
        vals_cp.wait()
        one_i = jnp.full((_L,), 1, jnp.int32)
        zero_i = jnp.full((_L,), 0, jnp.int32)

        def cstep(g, carry):
            ac, pc = carry
            x = nums_v[pl.ds(coff + g * _L, _L)]
            pres = x == x
            absn = jnp.logical_not(pres)
            v16 = vals_v[pl.ds(g * _L, _L)]
            pos = jnp.full((_L,), base + g * _L, jnp.int32) + lane
            m = jnp.where(absn, one_i, zero_i)
            p = m
            for k in (1, 2, 4, 8):
                sh = p.at[jnp.maximum(lane - k, 0)].get(
                    mode="promise_in_bounds")
                p = p + jnp.where(lane >= k, sh, zero_i)
            excl_a = p - m
            excl_p = lane - p + m
            slots_a = excl_a + ac
            slots_p = excl_p + pc
            plsc.store_scatter(cidx, [slots_a], v16, mask=absn)
            plsc.store_scatter(cposa, [slots_a], pos, mask=absn)
            plsc.store_scatter(cx, [slots_p], x, mask=pres)
            plsc.store_scatter(cposp, [slots_p], pos, mask=pres)
            na = p[_L - 1]
            ac = ac + na
            pc = pc + (_L - na)
            return ac, pc

        ac, pc = lax.fori_loop(0, _CHUNK // _L, cstep,
                               (jnp.int32(0), jnp.int32(0)))

        a0 = cidx[pl.ds(0, _L)]
        ap0 = cposa[pl.ds(0, _L)]
        px0 = cx[pl.ds(0, _L)]
        pp0 = cposp[pl.ds(0, _L)]
        cidx[pl.ds(ac, _L)] = jnp.full((_L,), a0[0])
        cposa[pl.ds(ac, _L)] = jnp.full((_L,), ap0[0])
        cx[pl.ds(pc, _L)] = jnp.full((_L,), px0[0])
        cposp[pl.ds(pc, _L)] = jnp.full((_L,), pp0[0])
        nba = (ac + _L - 1) // _L
        nbp = (pc + _L - 1) // _L

        for b in range(_NB):
            aidx2[b, pl.ds(0, _L)] = cidx[pl.ds(b * _L, _L)]

    with jax.named_scope("gfire"):
        def gfire_step(b, _):
            pltpu.async_copy(table_hbm.at[aidx2.at[b]],
                             rows_v.at[pl.ds(b * _L, _L)], gsem)
            return 0

        lax.fori_loop(0, nba, gfire_step, 0)

    with jax.named_scope("xchg"):
        plsc.subcore_barrier()
        pltpu.sync_copy(shared, all_v)
        tot = all_v[pl.ds(0, _L)]
        for j in range(1, _NS):
            tot = tot + all_v[pl.ds(j * _L, _L)]

    n = jnp.maximum(jnp.full((_L,), tot[2]), 1.0)
    mean_v = jnp.full((_L,), tot[0]) / n
    var_v = jnp.maximum(jnp.full((_L,), tot[1]) / n - mean_v * mean_v,
                        0.0) + _EPS
    bits = lax.bitcast_convert_type(var_v, jnp.int32)
    y = lax.bitcast_convert_type(0x5F3759DF - (bits >> 1), jnp.float32)
    for _ in range(4):
        y = y * (1.5 - 0.5 * var_v * y * y)
    gbv = gb_v[pl.ds(0, _L)]
    scale_v = y * jnp.full((_L,), gbv[0])
    beta_v = jnp.full((_L,), gbv[8])

    with jax.named_scope("poscopy"):
        for b in range(_NB):
            aposa2[b, pl.ds(0, _L)] = cposa[pl.ds(b * _L, _L)]
            aposp2[b, pl.ds(0, _L)] = cposp[pl.ds(b * _L, _L)]

    with jax.named_scope("pfill"):
        top = _TOT - nbp * _L

        def pfill(gg, _):
            xv = cx[pl.ds(gg * _L, _L)]
            nv = (xv - mean_v) * scale_v + beta_v
            rb = top + gg * _L
            for l in range(_L):
                sp = jnp.full((_L,), nv[l])
                for c in range(_D // _L):
                    rows_v[rb + l, pl.ds(c * _L, _L)] = sp
            pltpu.async_copy(rows_v.at[pl.ds(rb, _L)],
                             out_hbm.at[aposp2.at[gg]], osem)
            return 0

        lax.fori_loop(0, nbp, pfill, 0)

    with jax.named_scope("ascat"):
        def gdrain(b, _):
            pltpu.make_async_copy(table_hbm.at[aidx2.at[b]],
                                  rows_v.at[pl.ds(b * _L, _L)], gsem).wait()
            return 0

        lax.fori_loop(0, nba, gdrain, 0)

        def ascat(b, _):
            pltpu.async_copy(rows_v.at[pl.ds(b * _L, _L)],
                             out_hbm.at[aposa2.at[b]], osem)
            return 0

        lax.fori_loop(0, nba, ascat, 0)

    with jax.named_scope("odrain"):
        def odrain(b, _):
            pltpu.make_async_copy(rows_v.at[pl.ds(0, _L)],
                                  out_hbm.at[aposa2.at[0]], osem).wait()
            return 0

        lax.fori_loop(0, nba + nbp, odrain, 0)


@jax.jit
def _run(values, numbers, emb_table, gamma, beta):
    mesh = plsc.VectorSubcoreMesh(core_axis_name="c", subcore_axis_name="s",
                                  num_cores=_NC, num_subcores=_NS)
    return pl.kernel(
        _sc_body,
        out_type=jax.ShapeDtypeStruct((_N, _D), jnp.float32),
        mesh=mesh,
        compiler_params=pltpu.CompilerParams(needs_layout_passes=False),
        scratch_types=[
            pltpu.VMEM((_CHUNK,), jnp.int32),
            pltpu.VMEM((_SLICE,), jnp.float32),
            pltpu.VMEM((_TOT, _D), jnp.float32),
            pltpu.VMEM((_CB,), jnp.int32),
            pltpu.VMEM((_CB,), jnp.int32),
            pltpu.VMEM((_CB,), jnp.int32),
            pltpu.VMEM((_CB,), jnp.float32),
            pltpu.VMEM((_NB, _L), jnp.int32),
            pltpu.VMEM((_NB, _L), jnp.int32),
            pltpu.VMEM((_NB, _L), jnp.int32),
            pltpu.VMEM((_L,), jnp.float32),
            pltpu.VMEM((_L,), jnp.float32),
            pltpu.VMEM((_NS * _L,), jnp.float32),
            pltpu.VMEM_SHARED((_NS * _L,), jnp.float32),
            pltpu.SemaphoreType.DMA,
            pltpu.SemaphoreType.DMA,
            pltpu.SemaphoreType.DMA((2,)),
        ],
    )(values, numbers, emb_table, gamma, beta)


def kernel(values, numbers, emb_table, gamma, beta):
    return _run(values.astype(jnp.int32), numbers.astype(jnp.float32),
                emb_table, gamma.astype(jnp.float32),
                beta.astype(jnp.float32))


# speedup vs baseline: 1.0101x; 1.0101x over previous
import functools

import jax
import jax.numpy as jnp
from jax import lax
from jax.experimental import pallas as pl
from jax.experimental.pallas import tpu as pltpu
from jax.experimental.pallas import tpu_sc as plsc

_EPS = 1e-5
_N, _V, _D = 16384, 100000, 128
_NC, _NS, _L = 2, 16, 16
_NW = _NC * _NS
_CHUNK = _N // _NW
_SLICE = _N // _NS
_CB = _CHUNK + _L
_NB = _CB // _L
_TOT = _CHUNK + 2 * _L


def _sc_body(values_hbm, numbers_hbm, table_hbm, gamma_hbm, beta_hbm, out_hbm,
             vals_v, nums_v, rows_v, cidx, cposa, cposp, cx,
             aidx2, aposa2, aposp2, gb_v, pack_v, all_v, shared,
             gsem, osem, ssem):
    cid = lax.axis_index("c")
    sid = lax.axis_index("s")
    wid = sid * _NC + cid
    base = wid * _CHUNK

    with jax.named_scope("stage"):
        vals_cp = pltpu.async_copy(values_hbm.at[pl.ds(base, _CHUNK)],
                                   vals_v, ssem.at[0])
        num_cp = pltpu.async_copy(numbers_hbm.at[pl.ds(sid * _SLICE, _SLICE)],
                                  nums_v, ssem.at[1])
        g_cp = pltpu.async_copy(gamma_hbm, gb_v.at[pl.ds(0, 1)], ssem.at[1])
        b_cp = pltpu.async_copy(beta_hbm, gb_v.at[pl.ds(8, 1)], ssem.at[1])
        num_cp.wait()
        g_cp.wait()
        b_cp.wait()

    lane = lax.iota(jnp.int32, _L)

    def allsum(x):
        for k in (1, 2, 4, 8):
            x = x + x.at[lane ^ k].get(mode="promise_in_bounds")
        return x

    def stats_step(i, carry):
        s, ss, cnt = carry
        for u in range(8):
            x = nums_v[pl.ds(i * 8 * _L + u * _L, _L)]
            pres = x == x
            xs = jnp.where(pres, x, 0.0)
            s = s + xs
            ss = ss + xs * xs
            cnt = cnt + jnp.where(pres, 1.0, 0.0)
        return s, ss, cnt

    coff = cid * _CHUNK
    with jax.named_scope("compact"):
        vals_cp.wait()
        one_i = jnp.full((_L,), 1, jnp.int32)
        zero_i = jnp.full((_L,), 0, jnp.int32)

        def cstep(g, carry):
            ac, pc = carry
            x = nums_v[pl.ds(coff + g * _L, _L)]
            pres = x == x
            absn = jnp.logical_not(pres)
            v16 = vals_v[pl.ds(g * _L, _L)]
            pos = jnp.full((_L,), base + g * _L, jnp.int32) + lane
            m = jnp.where(absn, one_i, zero_i)
            p = m
            for k in (1, 2, 4, 8):
                sh = p.at[jnp.maximum(lane - k, 0)].get(
                    mode="promise_in_bounds")
                p = p + jnp.where(lane >= k, sh, zero_i)
            excl_a = p - m
            excl_p = lane - p + m
            slots_a = excl_a + ac
            slots_p = excl_p + pc
            plsc.store_scatter(cidx, [slots_a], v16, mask=absn)
            plsc.store_scatter(cposa, [slots_a], pos, mask=absn)
            plsc.store_scatter(cx, [slots_p], x, mask=pres)
            plsc.store_scatter(cposp, [slots_p], pos, mask=pres)
            na = p[_L - 1]
            ac = ac + na
            pc = pc + (_L - na)
            return ac, pc

        ac, pc = lax.fori_loop(0, _CHUNK // _L, cstep,
                               (jnp.int32(0), jnp.int32(0)))

        a0 = cidx[pl.ds(0, _L)]
        ap0 = cposa[pl.ds(0, _L)]
        px0 = cx[pl.ds(0, _L)]
        pp0 = cposp[pl.ds(0, _L)]
        cidx[pl.ds(ac, _L)] = jnp.full((_L,), a0[0])
        cposa[pl.ds(ac, _L)] = jnp.full((_L,), ap0[0])
        cx[pl.ds(pc, _L)] = jnp.full((_L,), px0[0])
        cposp[pl.ds(pc, _L)] = jnp.full((_L,), pp0[0])
        nba = (ac + _L - 1) // _L
        nbp = (pc + _L - 1) // _L

        for b in range(_NB):
            aidx2[b, pl.ds(0, _L)] = cidx[pl.ds(b * _L, _L)]

    with jax.named_scope("gfire"):
        def gfire_step(b, _):
            pltpu.async_copy(table_hbm.at[aidx2.at[b]],
                             rows_v.at[pl.ds(b * _L, _L)], gsem)
            return 0

        lax.fori_loop(0, nba, gfire_step, 0)

    with jax.named_scope("stats"):
        zero = jnp.zeros((_L,), jnp.float32)
        s, ss, cnt = lax.fori_loop(0, _SLICE // (8 * _L), stats_step,
                                   (zero, zero, zero))

    with jax.named_scope("xchg"):
        pack = jnp.where(lane == 0, allsum(s),
                         jnp.where(lane == 1, allsum(ss), allsum(cnt)))
        pack_v[pl.ds(0, _L)] = pack
        pltpu.sync_copy(pack_v, shared.at[pl.ds(sid * _L, _L)])
        plsc.subcore_barrier()
        pltpu.sync_copy(shared, all_v)
        tot = all_v[pl.ds(0, _L)]
        for j in range(1, _NS):
            tot = tot + all_v[pl.ds(j * _L, _L)]

    n = jnp.maximum(jnp.full((_L,), tot[2]), 1.0)
    mean_v = jnp.full((_L,), tot[0]) / n
    var_v = jnp.maximum(jnp.full((_L,), tot[1]) / n - mean_v * mean_v,
                        0.0) + _EPS
    bits = lax.bitcast_convert_type(var_v, jnp.int32)
    y = lax.bitcast_convert_type(0x5F3759DF - (bits >> 1), jnp.float32)
    for _ in range(4):
        y = y * (1.5 - 0.5 * var_v * y * y)
    gbv = gb_v[pl.ds(0, _L)]
    scale_v = y * jnp.full((_L,), gbv[0])
    beta_v = jnp.full((_L,), gbv[8])

    with jax.named_scope("poscopy"):
        for b in range(_NB):
            aposa2[b, pl.ds(0, _L)] = cposa[pl.ds(b * _L, _L)]
            aposp2[b, pl.ds(0, _L)] = cposp[pl.ds(b * _L, _L)]

    with jax.named_scope("pfill"):
        top = _TOT - nbp * _L

        def pfill(gg, _):
            xv = cx[pl.ds(gg * _L, _L)]
            nv = (xv - mean_v) * scale_v + beta_v
            rb = top + gg * _L
            for l in range(_L):
                sp = jnp.full((_L,), nv[l])
                for c in range(_D // _L):
                    rows_v[rb + l, pl.ds(c * _L, _L)] = sp
            pltpu.async_copy(rows_v.at[pl.ds(rb, _L)],
                             out_hbm.at[aposp2.at[gg]], osem)
            return 0

        lax.fori_loop(0, nbp, pfill, 0)

    with jax.named_scope("ascat"):
        def gdrain(b, _):
            pltpu.make_async_copy(table_hbm.at[aidx2.at[b]],
                                  rows_v.at[pl.ds(b * _L, _L)], gsem).wait()
            return 0

        lax.fori_loop(0, nba, gdrain, 0)

        def ascat(b, _):
            pltpu.async_copy(rows_v.at[pl.ds(b * _L, _L)],
                             out_hbm.at[aposa2.at[b]], osem)
            return 0

        lax.fori_loop(0, nba, ascat, 0)

    with jax.named_scope("odrain"):
        def odrain(b, _):
            pltpu.make_async_copy(rows_v.at[pl.ds(0, _L)],
                                  out_hbm.at[aposa2.at[0]], osem).wait()
            return 0

        lax.fori_loop(0, nba + nbp, odrain, 0)


@jax.jit
def _run(values, numbers, emb_table, gamma, beta):
    mesh = plsc.VectorSubcoreMesh(core_axis_name="c", subcore_axis_name="s",
                                  num_cores=_NC, num_subcores=_NS)
    return pl.kernel(
        _sc_body,
        out_type=jax.ShapeDtypeStruct((_N, _D), jnp.float32),
        mesh=mesh,
        compiler_params=pltpu.CompilerParams(needs_layout_passes=False),
        scratch_types=[
            pltpu.VMEM((_CHUNK,), jnp.int32),
            pltpu.VMEM((_SLICE,), jnp.float32),
            pltpu.VMEM((_TOT, _D), jnp.float32),
            pltpu.VMEM((_CB,), jnp.int32),
            pltpu.VMEM((_CB,), jnp.int32),
            pltpu.VMEM((_CB,), jnp.int32),
            pltpu.VMEM((_CB,), jnp.float32),
            pltpu.VMEM((_NB, _L), jnp.int32),
            pltpu.VMEM((_NB, _L), jnp.int32),
            pltpu.VMEM((_NB, _L), jnp.int32),
            pltpu.VMEM((_L,), jnp.float32),
            pltpu.VMEM((_L,), jnp.float32),
            pltpu.VMEM((_NS * _L,), jnp.float32),
            pltpu.VMEM_SHARED((_NS * _L,), jnp.float32),
            pltpu.SemaphoreType.DMA,
            pltpu.SemaphoreType.DMA,
            pltpu.SemaphoreType.DMA((2,)),
        ],
    )(values, numbers, emb_table, gamma, beta)


def kernel(values, numbers, emb_table, gamma, beta):
    return _run(values.astype(jnp.int32), numbers.astype(jnp.float32),
                emb_table, gamma.astype(jnp.float32),
                beta.astype(jnp.float32))


# speedup vs baseline: 1.0118x; 1.0016x over previous
import jax
import jax.numpy as jnp
from jax import lax
from jax.experimental import pallas as pl
from jax.experimental.pallas import tpu as pltpu
from jax.experimental.pallas import tpu_sc as plsc

_EPS = 1e-5
_N, _V, _D = 16384, 100000, 128
_NC, _NS, _L = 2, 16, 16
_NW = _NC * _NS
_CHUNK = _N // _NW
_SLICE = _N // _NS
_CB = _CHUNK + _L
_NB = _CB // _L
_TOT = _CHUNK + 2 * _L


def _sc_body(values_hbm, numbers_hbm, table_hbm, gamma_hbm, beta_hbm, out_hbm,
             vals_v, nums_v, rows_v, cidx, cposa, cposp, cx,
             aidx2, aposa2, aposp2, gb_v, pack_v, all_v, shared,
             gsem, osem, ssem):
    cid = lax.axis_index("c")
    sid = lax.axis_index("s")
    wid = sid * _NC + cid
    base = wid * _CHUNK

    with jax.named_scope("stage"):
        vals_cp = pltpu.async_copy(values_hbm.at[pl.ds(base, _CHUNK)],
                                   vals_v, ssem.at[0])
        num_cp = pltpu.async_copy(numbers_hbm.at[pl.ds(sid * _SLICE, _SLICE)],
                                  nums_v, ssem.at[1])
        g_cp = pltpu.async_copy(gamma_hbm, gb_v.at[pl.ds(0, 1)], ssem.at[1])
        b_cp = pltpu.async_copy(beta_hbm, gb_v.at[pl.ds(8, 1)], ssem.at[1])
        num_cp.wait()
        g_cp.wait()
        b_cp.wait()

    lane = lax.iota(jnp.int32, _L)

    def allsum(x):
        for k in (1, 2, 4, 8):
            x = x + x.at[lane ^ k].get(mode="promise_in_bounds")
        return x

    def stats_step(i, carry):
        s, ss, cnt = carry
        for u in range(8):
            x = nums_v[pl.ds(i * 8 * _L + u * _L, _L)]
            pres = x == x
            xs = jnp.where(pres, x, 0.0)
            s = s + xs
            ss = ss + xs * xs
            cnt = cnt + jnp.where(pres, 1.0, 0.0)
        return s, ss, cnt

    coff = cid * _CHUNK
    with jax.named_scope("compact"):
        vals_cp.wait()
        one_i = jnp.full((_L,), 1, jnp.int32)
        zero_i = jnp.full((_L,), 0, jnp.int32)

        def cstep(g, carry):
            ac, pc = carry
            x = nums_v[pl.ds(coff + g * _L, _L)]
            pres = x == x
            absn = jnp.logical_not(pres)
            v16 = vals_v[pl.ds(g * _L, _L)]
            pos = jnp.full((_L,), base + g * _L, jnp.int32) + lane
            m = jnp.where(absn, one_i, zero_i)
            p = m
            for k in (1, 2, 4, 8):
                sh = p.at[jnp.maximum(lane - k, 0)].get(
                    mode="promise_in_bounds")
                p = p + jnp.where(lane >= k, sh, zero_i)
            excl_a = p - m
            excl_p = lane - p + m
            slots_a = excl_a + ac
            slots_p = excl_p + pc
            plsc.store_scatter(cidx, [slots_a], v16, mask=absn)
            plsc.store_scatter(cposa, [slots_a], pos, mask=absn)
            plsc.store_scatter(cx, [slots_p], x, mask=pres)
            plsc.store_scatter(cposp, [slots_p], pos, mask=pres)
            na = p[_L - 1]
            ac = ac + na
            pc = pc + (_L - na)
            return ac, pc

        ac, pc = lax.fori_loop(0, _CHUNK // _L, cstep,
                               (jnp.int32(0), jnp.int32(0)))

        a0 = cidx[pl.ds(0, _L)]
        ap0 = cposa[pl.ds(0, _L)]
        px0 = cx[pl.ds(0, _L)]
        pp0 = cposp[pl.ds(0, _L)]
        cidx[pl.ds(ac, _L)] = jnp.full((_L,), a0[0])
        cposa[pl.ds(ac, _L)] = jnp.full((_L,), ap0[0])
        cx[pl.ds(pc, _L)] = jnp.full((_L,), px0[0])
        cposp[pl.ds(pc, _L)] = jnp.full((_L,), pp0[0])
        nba = (ac + _L - 1) // _L
        nbp = (pc + _L - 1) // _L

        for b in range(_NB):
            aidx2[b, pl.ds(0, _L)] = cidx[pl.ds(b * _L, _L)]

    with jax.named_scope("gfire"):
        def gfire_step(b, _):
            pltpu.async_copy(table_hbm.at[aidx2.at[b]],
                             rows_v.at[pl.ds(b * _L, _L)], gsem)
            return 0

        lax.fori_loop(0, nba, gfire_step, 0)

    with jax.named_scope("stats"):
        zero = jnp.zeros((_L,), jnp.float32)
        s, ss, cnt = lax.fori_loop(0, _SLICE // (8 * _L), stats_step,
                                   (zero, zero, zero))

    with jax.named_scope("xchg"):
        pack = jnp.where(lane == 0, allsum(s),
                         jnp.where(lane == 1, allsum(ss), allsum(cnt)))
        pack_v[pl.ds(0, _L)] = pack
        pltpu.sync_copy(pack_v, shared.at[pl.ds(sid * _L, _L)])
        plsc.subcore_barrier()
        pltpu.sync_copy(shared, all_v)
        tot = all_v[pl.ds(0, _L)]
        for j in range(1, _NS):
            tot = tot + all_v[pl.ds(j * _L, _L)]

    n = jnp.maximum(jnp.full((_L,), tot[2]), 1.0)
    mean_v = jnp.full((_L,), tot[0]) / n
    var_v = jnp.maximum(jnp.full((_L,), tot[1]) / n - mean_v * mean_v,
                        0.0) + _EPS
    bits = lax.bitcast_convert_type(var_v, jnp.int32)
    y = lax.bitcast_convert_type(0x5F3759DF - (bits >> 1), jnp.float32)
    for _ in range(4):
        y = y * (1.5 - 0.5 * var_v * y * y)
    gbv = gb_v[pl.ds(0, _L)]
    scale_v = y * jnp.full((_L,), gbv[0])
    beta_v = jnp.full((_L,), gbv[8])

    with jax.named_scope("poscopy"):
        for b in range(_NB):
            aposa2[b, pl.ds(0, _L)] = cposa[pl.ds(b * _L, _L)]
            aposp2[b, pl.ds(0, _L)] = cposp[pl.ds(b * _L, _L)]

    with jax.named_scope("pfill"):
        top = _TOT - nbp * _L

        def pfill(gg, _):
            xv = cx[pl.ds(gg * _L, _L)]
            nv = (xv - mean_v) * scale_v + beta_v
            rb = top + gg * _L
            for l in range(_L):
                sp = jnp.full((_L,), nv[l])
                for c in range(_D // _L):
                    rows_v[rb + l, pl.ds(c * _L, _L)] = sp
            pltpu.async_copy(rows_v.at[pl.ds(rb, _L)],
                             out_hbm.at[aposp2.at[gg]], osem)
            return 0

        lax.fori_loop(0, nbp, pfill, 0)

    with jax.named_scope("ascat"):
        def gdrain(b, _):
            pltpu.make_async_copy(table_hbm.at[aidx2.at[b]],
                                  rows_v.at[pl.ds(b * _L, _L)], gsem).wait()
            return 0

        lax.fori_loop(0, nba, gdrain, 0)

        def ascat(b, _):
            pltpu.async_copy(rows_v.at[pl.ds(b * _L, _L)],
                             out_hbm.at[aposa2.at[b]], osem)
            return 0

        lax.fori_loop(0, nba, ascat, 0)

    with jax.named_scope("odrain"):
        def odrain(b, _):
            pltpu.make_async_copy(rows_v.at[pl.ds(0, _L)],
                                  out_hbm.at[aposa2.at[0]], osem).wait()
            return 0

        lax.fori_loop(0, nba + nbp, odrain, 0)


@jax.jit
def _run(values, numbers, emb_table, gamma, beta):
    mesh = plsc.VectorSubcoreMesh(core_axis_name="c", subcore_axis_name="s",
                                  num_cores=_NC, num_subcores=_NS)
    return pl.kernel(
        _sc_body,
        out_type=jax.ShapeDtypeStruct((_N, _D), jnp.float32),
        mesh=mesh,
        compiler_params=pltpu.CompilerParams(needs_layout_passes=False),
        scratch_types=[
            pltpu.VMEM((_CHUNK,), jnp.int32),
            pltpu.VMEM((_SLICE,), jnp.float32),
            pltpu.VMEM((_TOT, _D), jnp.float32),
            pltpu.VMEM((_CB,), jnp.int32),
            pltpu.VMEM((_CB,), jnp.int32),
            pltpu.VMEM((_CB,), jnp.int32),
            pltpu.VMEM((_CB,), jnp.float32),
            pltpu.VMEM((_NB, _L), jnp.int32),
            pltpu.VMEM((_NB, _L), jnp.int32),
            pltpu.VMEM((_NB, _L), jnp.int32),
            pltpu.VMEM((_L,), jnp.float32),
            pltpu.VMEM((_L,), jnp.float32),
            pltpu.VMEM((_NS * _L,), jnp.float32),
            pltpu.VMEM_SHARED((_NS * _L,), jnp.float32),
            pltpu.SemaphoreType.DMA,
            pltpu.SemaphoreType.DMA,
            pltpu.SemaphoreType.DMA((2,)),
        ],
    )(values, numbers, emb_table, gamma, beta)


def kernel(values, numbers, emb_table, gamma, beta):
    return _run(values.astype(jnp.int32), numbers.astype(jnp.float32),
                emb_table, gamma.astype(jnp.float32),
                beta.astype(jnp.float32))
